# Initial kernel scaffold; baseline (speedup 1.0000x reference)
import jax
import jax.numpy as jnp
from jax.experimental import pallas as pl


def kernel(x, code_book):
    raise NotImplementedError("write your pallas kernel here")


# speedup vs baseline: 5.0989x; 5.0989x over previous
import jax
import jax.numpy as jnp
from jax.experimental import pallas as pl

_BLOCK = 2048


def _vq_block_kernel(ze_ref, cbt_ref, cb_ref, out_ref):
    ze = ze_ref[...]
    cbt = cbt_ref[...]
    cb = cb_ref[...]
    cb_norm = jnp.sum(cbt * cbt, axis=0)[None, :]
    dist = cb_norm - 2.0 * jax.lax.dot_general(
        ze, cbt, (((1,), (0,)), ((), ())), preferred_element_type=jnp.float32
    )
    num_emb = dist.shape[1]
    iota = jax.lax.broadcasted_iota(jnp.int32, dist.shape, 1)
    min_d = jnp.min(dist, axis=1, keepdims=True)
    idx = jnp.min(jnp.where(dist == min_d, iota, num_emb), axis=1, keepdims=True)
    onehot = (iota == idx).astype(jnp.float32)
    out_ref[...] = jax.lax.dot_general(
        onehot, cb, (((1,), (0,)), ((), ())), preferred_element_type=jnp.float32
    )


@jax.jit
def kernel(x, code_book):
    b, h, w, c = x.shape
    n = b * h * w
    ze = x.reshape(n, c)
    num_emb = code_book.shape[0]
    zq = pl.pallas_call(
        _vq_block_kernel,
        grid=(n // _BLOCK,),
        in_specs=[
            pl.BlockSpec((_BLOCK, c), lambda i: (i, 0)),
            pl.BlockSpec((c, num_emb), lambda i: (0, 0)),
            pl.BlockSpec((num_emb, c), lambda i: (0, 0)),
        ],
        out_specs=pl.BlockSpec((_BLOCK, c), lambda i: (i, 0)),
        out_shape=jax.ShapeDtypeStruct((n, c), x.dtype),
    )(ze, code_book.T, code_book)
    return zq.reshape(b, h, w, c)


# speedup vs baseline: 5.7925x; 1.1360x over previous
import jax
import jax.numpy as jnp
from jax.experimental import pallas as pl
from jax.experimental.pallas import tpu as pltpu

_BLOCK = 2048


def _vq_block_kernel(ze_ref, cbt_ref, cb_ref, out_ref):
    ze = ze_ref[...]
    cbt = cbt_ref[...]
    cb = cb_ref[...]
    cb_norm = jnp.sum(cbt * cbt, axis=0)[None, :]
    dist = cb_norm - 2.0 * jax.lax.dot_general(
        ze, cbt, (((1,), (0,)), ((), ())), preferred_element_type=jnp.float32
    )
    min_d = jnp.min(dist, axis=1, keepdims=True)
    hot = jnp.where(dist == min_d, 1.0, 0.0)
    count = jnp.sum(hot, axis=1, keepdims=True)
    zq = jax.lax.dot_general(
        hot, cb, (((1,), (0,)), ((), ())), preferred_element_type=jnp.float32
    )
    out_ref[...] = zq / count


@jax.jit
def kernel(x, code_book):
    b, h, w, c = x.shape
    n = b * h * w
    ze = x.reshape(n, c)
    num_emb = code_book.shape[0]
    zq = pl.pallas_call(
        _vq_block_kernel,
        grid=(n // _BLOCK,),
        in_specs=[
            pl.BlockSpec((_BLOCK, c), lambda i: (i, 0)),
            pl.BlockSpec((c, num_emb), lambda i: (0, 0)),
            pl.BlockSpec((num_emb, c), lambda i: (0, 0)),
        ],
        out_specs=pl.BlockSpec((_BLOCK, c), lambda i: (i, 0)),
        out_shape=jax.ShapeDtypeStruct((n, c), x.dtype),
        compiler_params=pltpu.CompilerParams(
            dimension_semantics=("parallel",),
        ),
    )(ze, code_book.T, code_book)
    return zq.reshape(b, h, w, c)


# speedup vs baseline: 6.2774x; 1.0837x over previous
import jax
import jax.numpy as jnp
from jax.experimental import pallas as pl
from jax.experimental.pallas import tpu as pltpu

_BLOCK = 4096


def _vq_block_kernel(ze_ref, cbt_ref, cb_ref, out_ref):
    ze = ze_ref[...]
    cbt = cbt_ref[...]
    cb = cb_ref[...]
    cb_norm = jnp.sum(cbt * cbt, axis=0)[None, :]
    dist = cb_norm - 2.0 * jax.lax.dot_general(
        ze, cbt, (((1,), (0,)), ((), ())), preferred_element_type=jnp.float32
    )
    min_d = jnp.min(dist, axis=1, keepdims=True)
    hot = jnp.where(dist == min_d, 1.0, 0.0)
    count = jnp.sum(hot, axis=1, keepdims=True)
    zq = jax.lax.dot_general(
        hot, cb, (((1,), (0,)), ((), ())), preferred_element_type=jnp.float32
    )
    out_ref[...] = zq / count


@jax.jit
def kernel(x, code_book):
    b, h, w, c = x.shape
    n = b * h * w
    ze = x.reshape(n, c)
    num_emb = code_book.shape[0]
    zq = pl.pallas_call(
        _vq_block_kernel,
        grid=(n // _BLOCK,),
        in_specs=[
            pl.BlockSpec((_BLOCK, c), lambda i: (i, 0)),
            pl.BlockSpec((c, num_emb), lambda i: (0, 0)),
            pl.BlockSpec((num_emb, c), lambda i: (0, 0)),
        ],
        out_specs=pl.BlockSpec((_BLOCK, c), lambda i: (i, 0)),
        out_shape=jax.ShapeDtypeStruct((n, c), x.dtype),
        compiler_params=pltpu.CompilerParams(
            dimension_semantics=("parallel",),
        ),
    )(ze, code_book.T, code_book)
    return zq.reshape(b, h, w, c)
